# baseline (speedup 1.0000x reference)
import functools

import jax
import jax.numpy as jnp
from jax import lax
from jax.experimental import pallas as pl
from jax.experimental.pallas import tpu as pltpu
from jax.experimental.pallas import tpu_sc as plsc

BATCH = 4096
MAX_LEN = 200
EMB = 64
TOTAL = BATCH * MAX_LEN
CHUNK = 128


def _make_gather(num_workers: int, nc: int):
    chunks_per_w = TOTAL // (CHUNK * num_workers)

    mesh = plsc.VectorSubcoreMesh(core_axis_name="c", subcore_axis_name="s")

    @functools.partial(
        pl.kernel,
        out_type=jax.ShapeDtypeStruct((TOTAL, EMB), jnp.float32),
        mesh=mesh,
        scratch_types=[
            pltpu.VMEM((chunks_per_w, CHUNK), jnp.int32),
            pltpu.VMEM((CHUNK, EMB), jnp.float32),
            pltpu.SemaphoreType.DMA,
        ],
        compiler_params=pltpu.CompilerParams(use_tc_tiling_on_sc=False),
    )
    def gather_kernel(table_hbm, idx_hbm, out_hbm, idx_v, rows_v, gsem):
        wid = lax.axis_index("s") * nc + lax.axis_index("c")
        row0 = wid * chunks_per_w
        pltpu.sync_copy(idx_hbm.at[pl.ds(row0, chunks_per_w)], idx_v)

        def body(j, carry):
            pltpu.async_copy(table_hbm.at[idx_v.at[j]], rows_v, gsem).wait()
            pltpu.sync_copy(
                rows_v, out_hbm.at[pl.ds((row0 + j) * CHUNK, CHUNK)]
            )
            return carry

        lax.fori_loop(0, chunks_per_w, body, 0)

    return gather_kernel


def kernel(x, W):
    info = plsc.get_sparse_core_info()
    nw = info.num_cores * info.num_subcores
    idx = x.reshape(TOTAL // CHUNK, CHUNK)
    out = _make_gather(nw, info.num_cores)(W, idx)
    return out.reshape(BATCH, MAX_LEN, EMB)


# speedup vs baseline: 1.1161x; 1.1161x over previous
import functools

import jax
import jax.numpy as jnp
from jax import lax
from jax.experimental import pallas as pl
from jax.experimental.pallas import tpu as pltpu
from jax.experimental.pallas import tpu_sc as plsc

BATCH = 4096
MAX_LEN = 200
EMB = 64
TOTAL = BATCH * MAX_LEN
CHUNK = 128


NBUF = 8


def _make_gather(num_workers: int, nc: int):
    chunks_per_w = TOTAL // (CHUNK * num_workers)
    nsteps = chunks_per_w // NBUF

    mesh = plsc.VectorSubcoreMesh(core_axis_name="c", subcore_axis_name="s")

    @functools.partial(
        pl.kernel,
        out_type=jax.ShapeDtypeStruct((TOTAL, EMB), jnp.float32),
        mesh=mesh,
        scratch_types=[
            pltpu.VMEM((chunks_per_w, CHUNK), jnp.int32),
            [pltpu.VMEM((CHUNK, EMB), jnp.float32) for _ in range(NBUF)],
            pltpu.SemaphoreType.DMA((NBUF,)),
            pltpu.SemaphoreType.DMA((NBUF,)),
        ],
        compiler_params=pltpu.CompilerParams(use_tc_tiling_on_sc=False),
    )
    def gather_kernel(table_hbm, idx_hbm, out_hbm, idx_v, rows, gsem, osem):
        wid = lax.axis_index("s") * nc + lax.axis_index("c")
        row0 = wid * chunks_per_w
        pltpu.sync_copy(idx_hbm.at[pl.ds(row0, chunks_per_w)], idx_v)

        def start_gather(g, b):
            pltpu.async_copy(table_hbm.at[idx_v.at[g]], rows[b], gsem.at[b])

        def start_out(g, b):
            pltpu.async_copy(
                rows[b], out_hbm.at[pl.ds((row0 + g) * CHUNK, CHUNK)],
                osem.at[b],
            )

        def wait_gather(b):
            pltpu.make_async_copy(
                table_hbm.at[idx_v.at[0]], rows[b], gsem.at[b]
            ).wait()

        def wait_out(b):
            pltpu.make_async_copy(
                rows[b], out_hbm.at[pl.ds(0, CHUNK)], osem.at[b]
            ).wait()

        for b in range(NBUF):
            start_gather(b, b)

        def body(i, carry):
            g0 = i * NBUF
            for b in range(NBUF):
                wait_gather(b)
                start_out(g0 + b, b)
            for b in range(NBUF):
                wait_out(b)
                start_gather(g0 + NBUF + b, b)
            return carry

        lax.fori_loop(0, nsteps - 1, body, 0)

        g0 = (nsteps - 1) * NBUF
        for b in range(NBUF):
            wait_gather(b)
            start_out(g0 + b, b)
        for b in range(NBUF):
            wait_out(b)

    return gather_kernel


def kernel(x, W):
    info = plsc.get_sparse_core_info()
    nw = info.num_cores * info.num_subcores
    idx = x.reshape(TOTAL // CHUNK, CHUNK)
    out = _make_gather(nw, info.num_cores)(W, idx)
    return out.reshape(BATCH, MAX_LEN, EMB)


# speedup vs baseline: 1.3587x; 1.2174x over previous
import functools

import jax
import jax.numpy as jnp
from jax import lax
from jax.experimental import pallas as pl
from jax.experimental.pallas import tpu as pltpu
from jax.experimental.pallas import tpu_sc as plsc

BATCH = 4096
MAX_LEN = 200
EMB = 64
PAD_EMB = 128
TOTAL = BATCH * MAX_LEN
CHUNK = 128
NBUF = 4


def _make_gather(num_workers: int, nc: int):
    chunks_per_w = TOTAL // (CHUNK * num_workers)
    nsteps = chunks_per_w // NBUF

    mesh = plsc.VectorSubcoreMesh(core_axis_name="c", subcore_axis_name="s")

    @functools.partial(
        pl.kernel,
        out_type=jax.ShapeDtypeStruct((TOTAL, PAD_EMB), jnp.float32),
        mesh=mesh,
        scratch_types=[
            pltpu.VMEM((chunks_per_w, CHUNK), jnp.int32),
            [pltpu.VMEM((CHUNK, PAD_EMB), jnp.float32) for _ in range(NBUF)],
            pltpu.SemaphoreType.DMA((NBUF,)),
            pltpu.SemaphoreType.DMA((NBUF,)),
        ],
    )
    def gather_kernel(table_hbm, idx_hbm, out_hbm, idx_v, rows, gsem, osem):
        wid = lax.axis_index("s") * nc + lax.axis_index("c")
        row0 = wid * chunks_per_w
        pltpu.sync_copy(idx_hbm.at[pl.ds(row0, chunks_per_w)], idx_v)

        def start_gather(g, b):
            pltpu.async_copy(table_hbm.at[idx_v.at[g]], rows[b], gsem.at[b])

        def start_out(g, b):
            pltpu.async_copy(
                rows[b],
                out_hbm.at[pl.ds((row0 + g) * CHUNK, CHUNK)],
                osem.at[b],
            )

        def wait_gather(b):
            pltpu.make_async_copy(
                table_hbm.at[idx_v.at[0]], rows[b], gsem.at[b]
            ).wait()

        def wait_out(b):
            pltpu.make_async_copy(
                rows[b], out_hbm.at[pl.ds(0, CHUNK)], osem.at[b]
            ).wait()

        for b in range(NBUF):
            start_gather(b, b)

        def body(i, carry):
            g0 = i * NBUF
            for b in range(NBUF):
                wait_gather(b)
                start_out(g0 + b, b)
            for b in range(NBUF):
                wait_out(b)
                start_gather(g0 + NBUF + b, b)
            return carry

        lax.fori_loop(0, nsteps - 1, body, 0)

        g0 = (nsteps - 1) * NBUF
        for b in range(NBUF):
            wait_gather(b)
            start_out(g0 + b, b)
        for b in range(NBUF):
            wait_out(b)

    return gather_kernel


def kernel(x, W):
    info = plsc.get_sparse_core_info()
    nw = info.num_cores * info.num_subcores
    W128 = jnp.pad(W, ((0, 0), (0, PAD_EMB - EMB)))
    idx = x.reshape(TOTAL // CHUNK, CHUNK)
    out = _make_gather(nw, info.num_cores)(W128, idx)
    return out[:, :EMB].reshape(BATCH, MAX_LEN, EMB)
